# baseline (speedup 1.0000x reference)
import jax
import jax.numpy as jnp
from jax import lax
from jax.experimental import pallas as pl
from jax.experimental.pallas import tpu as pltpu
from jax.experimental.pallas import tpu_sc as plsc

NC, NS, LANES = 2, 16, 16
NW = NC * NS
LT = 128
LTP = LT + 1
NBUF = 4
LOOKAHEAD = 2


def _emb_body(x_hbm, tab_hbm, pos_hbm, out_hbm,
              i0, i1, i2, i3, r0, r1, r2, r3, t0, t1, t2, t3, pos_v,
              g0, g1, g2, g3, w0, w1, w2, w3):
    idx = [i0, i1, i2, i3]
    rows = [r0, r1, r2, r3]
    pads = [t0, t1, t2, t3]
    gsem = [g0, g1, g2, g3]
    wsem = [w0, w1, w2, w3]

    wid = lax.axis_index("s") * NC + lax.axis_index("c")
    batch = out_hbm.shape[0]
    seq = pos_hbm.shape[0]
    emb = tab_hbm.shape[1]
    ltiles = seq // LT
    batches_per_w = batch // NW
    nblocks = batches_per_w * ltiles
    base_b = wid * batches_per_w

    iota = lax.iota(jnp.int32, LANES)
    ehi = [(iota + k * LANES) // 8 for k in range(emb // LANES)]
    elo = [(iota + k * LANES) % 8 for k in range(emb // LANES)]

    pltpu.sync_copy(pos_hbm, pos_v)

    def transpose_add(src, dst, pos_off):
        @plsc.parallel_loop(0, LT, unroll=2)
        def l_body(l):
            lsplat = jnp.full((LANES,), l, dtype=jnp.int32)
            for k in range(emb // LANES):
                sl = pl.ds(k * LANES, LANES)
                vals = src[l, sl] + pos_v[pos_off + l, sl]
                plsc.store_scatter(dst, [ehi[k], elo[k], lsplat], vals)

    def block_coords(q, r):
        return base_b + q, r

    def load_idx(bb, lt, slot):
        pltpu.sync_copy(x_hbm.at[bb // 8, lt, bb % 8], idx[slot])

    for b in range(LOOKAHEAD):
        bb, lt = block_coords(0, b)
        load_idx(bb, lt, b)
        pltpu.async_copy(tab_hbm.at[idx[b]], rows[b], gsem[b])

    def ring(i, carry):
        for b in range(NBUF):
            c = i * NBUF + b
            s = b
            sp = (b + LOOKAHEAD) % NBUF

            @pl.when(c + LOOKAHEAD < nblocks)
            def _():
                @pl.when(c >= NBUF - LOOKAHEAD)
                def _():
                    wb, wlt = block_coords(i + (b + LOOKAHEAD - NBUF) // NBUF,
                                           sp)
                    pltpu.make_async_copy(
                        pads[sp].at[:, :, pl.ds(0, LT)],
                        out_hbm.at[wb, :, wlt], wsem[sp]).wait()
                gb, glt = block_coords(i + (b + LOOKAHEAD) // NBUF, sp)
                load_idx(gb, glt, sp)
                pltpu.async_copy(tab_hbm.at[idx[sp]], rows[sp], gsem[sp])

            pltpu.make_async_copy(tab_hbm.at[idx[s]], rows[s], gsem[s]).wait()
            bb, lt = block_coords(i, b)
            transpose_add(rows[s], pads[s], lt * LT)
            pltpu.async_copy(pads[s].at[:, :, pl.ds(0, LT)],
                             out_hbm.at[bb, :, lt], wsem[s])
        return carry

    lax.fori_loop(0, nblocks // NBUF, ring, 0)

    for b in range(NBUF):
        bb, lt = block_coords(nblocks // NBUF - 1, b)
        pltpu.make_async_copy(pads[b].at[:, :, pl.ds(0, LT)],
                              out_hbm.at[bb, :, lt], wsem[b]).wait()


def kernel(x, embed_w, pos_w):
    batch, seq = x.shape
    _, emb = embed_w.shape
    et, ltiles = emb // 8, seq // LT
    mesh = plsc.VectorSubcoreMesh(
        core_axis_name="c", subcore_axis_name="s",
        num_cores=NC, num_subcores=NS,
    )
    xphys = x.reshape(batch // 8, 8, ltiles, LT).transpose(0, 2, 1, 3)
    phys = pl.kernel(
        _emb_body,
        out_type=jax.ShapeDtypeStruct((batch, et, ltiles, 8, LT), jnp.float32),
        mesh=mesh,
        compiler_params=pltpu.CompilerParams(
            use_tc_tiling_on_sc=False, needs_layout_passes=False),
        scratch_types=(
            [pltpu.VMEM((LT,), jnp.int32) for _ in range(NBUF)]
            + [pltpu.VMEM((LT, emb), jnp.float32) for _ in range(NBUF)]
            + [pltpu.VMEM((et, 8, LTP), jnp.float32) for _ in range(NBUF)]
            + [pltpu.VMEM((seq, emb), jnp.float32)]
            + [pltpu.SemaphoreType.DMA for _ in range(2 * NBUF)]
        ),
    )(xphys, embed_w, pos_w)
    return phys.transpose(0, 2, 4, 1, 3).reshape(batch, seq, emb)


# speedup vs baseline: 1.4042x; 1.4042x over previous
import jax
import jax.numpy as jnp
from jax import lax
from jax.experimental import pallas as pl
from jax.experimental.pallas import tpu as pltpu
from jax.experimental.pallas import tpu_sc as plsc

NC, NS, LANES = 2, 16, 16
NW = NC * NS
LT = 128
LTP = LT + 1
NBUF = 4
LOOKAHEAD = 2


def _emb_body(x_hbm, tab_hbm, pos_hbm, out_hbm,
              idx_all, r0, r1, r2, r3, t0, t1, t2, t3, pos_v,
              g0, g1, g2, g3, w0, w1, w2, w3):
    rows = [r0, r1, r2, r3]
    pads = [t0, t1, t2, t3]
    gsem = [g0, g1, g2, g3]
    wsem = [w0, w1, w2, w3]

    wid = lax.axis_index("s") * NC + lax.axis_index("c")
    batch = out_hbm.shape[0]
    seq = pos_hbm.shape[0]
    emb = tab_hbm.shape[1]
    ltiles = seq // LT
    batches_per_w = batch // NW
    nblocks = batches_per_w * ltiles
    base_b = wid * batches_per_w

    iota = lax.iota(jnp.int32, LANES)
    ehi = [(iota + k * LANES) // 8 for k in range(emb // LANES)]
    elo = [(iota + k * LANES) % 8 for k in range(emb // LANES)]

    pltpu.sync_copy(pos_hbm, pos_v)

    def transpose_add(src, dst, pos_off):
        @plsc.parallel_loop(0, LT, unroll=4)
        def l_body(l):
            lsplat = jnp.full((LANES,), l, dtype=jnp.int32)
            for k in range(emb // LANES):
                sl = pl.ds(k * LANES, LANES)
                vals = src[l, sl] + pos_v[pos_off + l, sl]
                plsc.store_scatter(dst, [ehi[k], elo[k], lsplat], vals)

    def block_coords(q, r):
        return base_b + q, r

    pltpu.sync_copy(x_hbm.at[pl.ds(base_b // 8, batches_per_w // 8)], idx_all)

    def start_gather(q, lt, slot):
        pltpu.async_copy(tab_hbm.at[idx_all.at[q // 8, lt, q % 8]],
                         rows[slot], gsem[slot])

    for b in range(LOOKAHEAD):
        start_gather(0, b, b)

    def ring(i, carry):
        for b in range(NBUF):
            c = i * NBUF + b
            s = b
            sp = (b + LOOKAHEAD) % NBUF

            @pl.when(c + LOOKAHEAD < nblocks)
            def _():
                @pl.when(c >= NBUF - LOOKAHEAD)
                def _():
                    wb, wlt = block_coords(i + (b + LOOKAHEAD - NBUF) // NBUF,
                                           sp)
                    pltpu.make_async_copy(
                        pads[sp].at[:, :, pl.ds(0, LT)],
                        out_hbm.at[wb, :, wlt], wsem[sp]).wait()
                start_gather(i + (b + LOOKAHEAD) // NBUF, sp, sp)

            pltpu.make_async_copy(tab_hbm.at[idx_all.at[i // 8, b, i % 8]],
                                  rows[s], gsem[s]).wait()
            bb, lt = block_coords(i, b)
            transpose_add(rows[s], pads[s], lt * LT)
            pltpu.async_copy(pads[s].at[:, :, pl.ds(0, LT)],
                             out_hbm.at[bb, :, lt], wsem[s])
        return carry

    lax.fori_loop(0, nblocks // NBUF, ring, 0)

    for b in range(NBUF):
        bb, lt = block_coords(nblocks // NBUF - 1, b)
        pltpu.make_async_copy(pads[b].at[:, :, pl.ds(0, LT)],
                              out_hbm.at[bb, :, lt], wsem[b]).wait()


def kernel(x, embed_w, pos_w):
    batch, seq = x.shape
    _, emb = embed_w.shape
    et, ltiles = emb // 8, seq // LT
    mesh = plsc.VectorSubcoreMesh(
        core_axis_name="c", subcore_axis_name="s",
        num_cores=NC, num_subcores=NS,
    )
    xphys = x.reshape(batch // 8, 8, ltiles, LT).transpose(0, 2, 1, 3)
    phys = pl.kernel(
        _emb_body,
        out_type=jax.ShapeDtypeStruct((batch, et, ltiles, 8, LT), jnp.float32),
        mesh=mesh,
        compiler_params=pltpu.CompilerParams(
            use_tc_tiling_on_sc=False, needs_layout_passes=False),
        scratch_types=(
            [pltpu.VMEM((batch // NW // 8, ltiles, 8, LT), jnp.int32)]
            + [pltpu.VMEM((LT, emb), jnp.float32) for _ in range(NBUF)]
            + [pltpu.VMEM((et, 8, LTP), jnp.float32) for _ in range(NBUF)]
            + [pltpu.VMEM((seq, emb), jnp.float32)]
            + [pltpu.SemaphoreType.DMA for _ in range(2 * NBUF)]
        ),
    )(xphys, embed_w, pos_w)
    return phys.transpose(0, 2, 4, 1, 3).reshape(batch, seq, emb)
